# baseline (speedup 1.0000x reference)
import jax
import jax.numpy as jnp
from jax.experimental import pallas as pl
from jax.experimental.pallas import tpu as pltpu

_B = 32768
_D = 2048
_E = 64
_BLK = 1024


def _router_body(rv_ref, emb_ref, idx_ref, scores_ref):
    emb = emb_ref[...]
    norms = jnp.clip(jnp.sqrt(jnp.sum(emb * emb, axis=1, keepdims=True)), 1e-12)
    normed = emb / norms
    scores = jax.lax.dot_general(
        rv_ref[...], normed,
        dimension_numbers=(((1,), (1,)), ((), ())),
        preferred_element_type=jnp.float32,
    )
    scores_ref[...] = scores
    idx_ref[...] = jnp.argmax(scores, axis=1).astype(jnp.int32)


def kernel(route_vec, expert_embeddings):
    grid = (_B // _BLK,)
    idx, scores = pl.pallas_call(
        _router_body,
        grid=grid,
        in_specs=[
            pl.BlockSpec((_BLK, _D), lambda i: (i, 0)),
            pl.BlockSpec((_E, _D), lambda i: (0, 0)),
        ],
        out_specs=[
            pl.BlockSpec((_BLK,), lambda i: (i,)),
            pl.BlockSpec((_BLK, _E), lambda i: (i, 0)),
        ],
        out_shape=[
            jax.ShapeDtypeStruct((_B,), jnp.int32),
            jax.ShapeDtypeStruct((_B, _E), jnp.float32),
        ],
        compiler_params=pltpu.CompilerParams(
            dimension_semantics=("parallel",),
        ),
    )(route_vec, expert_embeddings)
    return (idx, scores)


# speedup vs baseline: 1.0492x; 1.0492x over previous
import jax
import jax.numpy as jnp
from jax.experimental import pallas as pl
from jax.experimental.pallas import tpu as pltpu

_B = 32768
_D = 2048
_E = 64
_BLK = 2048


def _prep_body(emb_ref, normed_t_ref):
    emb = emb_ref[...]
    norms = jnp.clip(jnp.sqrt(jnp.sum(emb * emb, axis=1, keepdims=True)), 1e-12)
    normed_t_ref[...] = (emb / norms).T


def _router_body(rv_ref, wt_ref, idx_ref, scores_ref):
    scores = jax.lax.dot_general(
        rv_ref[...], wt_ref[...],
        dimension_numbers=(((1,), (0,)), ((), ())),
        preferred_element_type=jnp.float32,
    )
    scores_ref[...] = scores
    idx_ref[...] = jnp.argmax(scores, axis=1).astype(jnp.int32)


def kernel(route_vec, expert_embeddings):
    normed_t = pl.pallas_call(
        _prep_body,
        out_shape=jax.ShapeDtypeStruct((_D, _E), jnp.float32),
    )(expert_embeddings)
    grid = (_B // _BLK,)
    idx, scores = pl.pallas_call(
        _router_body,
        grid=grid,
        in_specs=[
            pl.BlockSpec((_BLK, _D), lambda i: (i, 0)),
            pl.BlockSpec((_D, _E), lambda i: (0, 0)),
        ],
        out_specs=[
            pl.BlockSpec((_BLK,), lambda i: (i,)),
            pl.BlockSpec((_BLK, _E), lambda i: (i, 0)),
        ],
        out_shape=[
            jax.ShapeDtypeStruct((_B,), jnp.int32),
            jax.ShapeDtypeStruct((_B, _E), jnp.float32),
        ],
        compiler_params=pltpu.CompilerParams(
            dimension_semantics=("parallel",),
        ),
    )(route_vec, normed_t)
    return (idx, scores)


# speedup vs baseline: 1.0675x; 1.0174x over previous
import jax
import jax.numpy as jnp
from jax.experimental import pallas as pl
from jax.experimental.pallas import tpu as pltpu

_B = 32768
_D = 2048
_E = 64
_BLK = 2048


def _router_body(rv_ref, emb_ref, idx_ref, scores_ref, wt_ref):
    @pl.when(pl.program_id(0) == 0)
    def _prep():
        emb = emb_ref[...]
        norms = jnp.clip(jnp.sqrt(jnp.sum(emb * emb, axis=1, keepdims=True)), 1e-12)
        wt_ref[...] = (emb / norms).T

    scores = jax.lax.dot_general(
        rv_ref[...], wt_ref[...],
        dimension_numbers=(((1,), (0,)), ((), ())),
        preferred_element_type=jnp.float32,
    )
    scores_ref[...] = scores
    idx_ref[...] = jnp.argmax(scores, axis=1).astype(jnp.int32)


def kernel(route_vec, expert_embeddings):
    grid = (_B // _BLK,)
    idx, scores = pl.pallas_call(
        _router_body,
        grid=grid,
        in_specs=[
            pl.BlockSpec((_BLK, _D), lambda i: (i, 0)),
            pl.BlockSpec((_E, _D), lambda i: (0, 0)),
        ],
        out_specs=[
            pl.BlockSpec((_BLK,), lambda i: (i,)),
            pl.BlockSpec((_BLK, _E), lambda i: (i, 0)),
        ],
        out_shape=[
            jax.ShapeDtypeStruct((_B,), jnp.int32),
            jax.ShapeDtypeStruct((_B, _E), jnp.float32),
        ],
        scratch_shapes=[pltpu.VMEM((_D, _E), jnp.float32)],
        compiler_params=pltpu.CompilerParams(
            dimension_semantics=("arbitrary",),
        ),
    )(route_vec, expert_embeddings)
    return (idx, scores)
